# Initial kernel scaffold; baseline (speedup 1.0000x reference)
import jax
import jax.numpy as jnp
from jax.experimental import pallas as pl


def kernel(x1, lu_idx, lu_vals, ld_idx, ld_vals, batch1, Wp1, Wg1, asrc1, adst1, Wp2, Wg2, asrc2, adst2, Wp3, Wg3, asrc3, adst3, Wp4, Wg4, asrc4, adst4):
    raise NotImplementedError("write your pallas kernel here")


# speedup vs baseline: 4.2850x; 4.2850x over previous
import functools
import jax
import jax.numpy as jnp
from jax import lax
from jax.experimental import pallas as pl
from jax.experimental.pallas import tpu as pltpu
from jax.experimental.pallas import tpu_sc as plsc

N = 10000
E = 320000
NB = 64
FW = 32

NC = 2
NS = 16
NW = NC * NS


def _mesh():
    return plsc.VectorSubcoreMesh(core_axis_name="c", subcore_axis_name="s")


def _sc_gather(table, gidx):
    M = gidx.shape[0]
    W = table.shape[1]
    per = M // NW
    CH = 80
    nch = per // CH

    @functools.partial(
        pl.kernel, mesh=_mesh(),
        compiler_params=pltpu.CompilerParams(use_tc_tiling_on_sc=False),
        out_type=jax.ShapeDtypeStruct((M, W), jnp.float32),
        scratch_types=[
            pltpu.VMEM((CH,), jnp.int32),
            pltpu.VMEM((CH, W), jnp.float32),
            pltpu.SemaphoreType.DMA,
        ],
    )
    def k(table_h, idx_h, out_h, idxv, rowsv, sem):
        wid = lax.axis_index("s") * NC + lax.axis_index("c")
        base = wid * per

        def body(c, carry):
            o = base + c * CH
            pltpu.sync_copy(idx_h.at[pl.ds(o, CH)], idxv)
            pltpu.async_copy(table_h.at[idxv], rowsv, sem).wait()
            pltpu.sync_copy(rowsv, out_h.at[pl.ds(o, CH)])
            return carry

        lax.fori_loop(0, nch, body, 0)

    return k(table, gidx)


def _tc_elemwise(body, out_cols, ins):
    M = ins[0].shape[0]
    BM = 8000
    grid = M // BM
    return pl.pallas_call(
        body,
        grid=(grid,),
        in_specs=[pl.BlockSpec((BM, a.shape[1]), lambda i: (i, 0)) for a in ins],
        out_specs=pl.BlockSpec((BM, out_cols), lambda i: (i, 0)),
        out_shape=jax.ShapeDtypeStruct((M, out_cols), jnp.float32),
    )(*ins)


def _tc_logits(GA, GB):
    def body(a_ref, b_ref, o_ref):
        v = a_ref[:, 0:1] + b_ref[:, 1:2]
        o_ref[...] = jnp.maximum(v, 0.2 * v)
    return _tc_elemwise(body, 1, [GA, GB])


def _tc_exp(e, GM):
    def body(e_ref, m_ref, o_ref):
        o_ref[...] = jnp.exp(e_ref[...] - m_ref[:, 0:1])
    return _tc_elemwise(body, 1, [e, GM])


def _tc_pv(vals, GD1, GD2):
    def body(v_ref, a_ref, b_ref, o_ref):
        o_ref[...] = v_ref[...] * a_ref[:, 0:1] * b_ref[:, 0:1]
    return _tc_elemwise(body, 1, [vals, GD1, GD2])


def _pad16(col):
    return jnp.concatenate([col[:, None], jnp.zeros((col.shape[0], 15), jnp.float32)], axis=1)


def _sc_scatter(vals48, sidx, nrows):
    M = sidx.shape[0]
    per = M // NW
    CH = 80
    nch = per // CH
    rps = nrows // NS
    ZR = 625
    nz = rps // ZR

    @functools.partial(
        pl.kernel, mesh=_mesh(),
        compiler_params=pltpu.CompilerParams(use_tc_tiling_on_sc=False),
        out_type=jax.ShapeDtypeStruct((NC, nrows, 48), jnp.float32),
        scratch_types=[
            pltpu.VMEM((CH,), jnp.int32),
            pltpu.VMEM((CH, 48), jnp.float32),
            pltpu.VMEM((ZR, 48), jnp.float32),
            pltpu.VMEM_SHARED((nrows, 48), jnp.float32),
        ],
    )
    def k(v_h, i_h, out_h, idxv, valv, zbuf, acc):
        c = lax.axis_index("c")
        s = lax.axis_index("s")
        wid = s * NC + c
        base = wid * per
        r0 = s * rps

        def zb(i, carry):
            zbuf[i, pl.ds(0, 16)] = jnp.zeros((16,), jnp.float32)
            zbuf[i, pl.ds(16, 16)] = jnp.zeros((16,), jnp.float32)
            zbuf[i, pl.ds(32, 16)] = jnp.zeros((16,), jnp.float32)
            return carry

        lax.fori_loop(0, ZR, zb, 0)
        for j in range(nz):
            pltpu.sync_copy(zbuf, acc.at[pl.ds(r0 + j * ZR, ZR)])
        plsc.subcore_barrier()

        def body(ci, carry):
            o = base + ci * CH
            pltpu.sync_copy(i_h.at[pl.ds(o, CH)], idxv)
            pltpu.sync_copy(v_h.at[pl.ds(o, CH)], valv)
            pltpu.sync_copy(valv, acc.at[idxv], add=True)
            return carry

        lax.fori_loop(0, nch, body, 0)
        plsc.subcore_barrier()
        pltpu.sync_copy(acc.at[pl.ds(r0, rps)], out_h.at[c, pl.ds(r0, rps)])

    return k(vals48, sidx)


def _tc_dense(x_or_sc, Wg, Wp, a_s, a_d, first):
    BN = 1000
    grid = N // BN

    def body(in_ref, wg_ref, wp_ref, as_ref, ad_ref, T_ref, es_ref, ed_ref):
        if first:
            x = in_ref[...]
        else:
            A = in_ref[...]
            num = A[0] + A[1]
            hu = num[0, :, :32] / (num[0, :, 32:33] + 1e-16)
            hd = num[1, :, :32] / (num[1, :, 32:33] + 1e-16)
            hp = num[2, :, :32]
            x = jnp.maximum(hu + hd + hp, 0.0)
        hg = jnp.dot(x, wg_ref[...], preferred_element_type=jnp.float32)
        hpj = jnp.dot(x, wp_ref[...], preferred_element_type=jnp.float32)
        T_ref[0] = hg
        T_ref[1] = hpj
        es_ref[...] = jnp.sum(hg * as_ref[...], axis=1, keepdims=True)
        ed_ref[...] = jnp.sum(hg * ad_ref[...], axis=1, keepdims=True)

    din = x_or_sc.shape[-1] if first else 32
    in_spec = (
        pl.BlockSpec((BN, din), lambda i: (i, 0))
        if first
        else pl.BlockSpec((NC, 3, BN, 48), lambda i: (0, 0, i, 0))
    )
    T, es, ed = pl.pallas_call(
        body,
        grid=(grid,),
        in_specs=[
            in_spec,
            pl.BlockSpec((din if first else 32, 32), lambda i: (0, 0)),
            pl.BlockSpec((din if first else 32, 32), lambda i: (0, 0)),
            pl.BlockSpec((1, 32), lambda i: (0, 0)),
            pl.BlockSpec((1, 32), lambda i: (0, 0)),
        ],
        out_specs=[
            pl.BlockSpec((2, BN, 32), lambda i: (0, i, 0)),
            pl.BlockSpec((BN, 1), lambda i: (i, 0)),
            pl.BlockSpec((BN, 1), lambda i: (i, 0)),
        ],
        out_shape=[
            jax.ShapeDtypeStruct((2, N, 32), jnp.float32),
            jax.ShapeDtypeStruct((N, 1), jnp.float32),
            jax.ShapeDtypeStruct((N, 1), jnp.float32),
        ],
    )(x_or_sc, Wg, Wp, a_s, a_d)
    return T.reshape(2 * N, 32), es, ed


def _tc_scale(R, w):
    M = R.shape[0]
    BM = 8000
    grid = M // BM

    def body(r_ref, w_ref, o_ref):
        r = r_ref[...]
        ww = w_ref[...]
        o_ref[:, :32] = r * ww
        o_ref[:, 32:33] = ww
        o_ref[:, 33:] = jnp.zeros((BM, 15), jnp.float32)

    return pl.pallas_call(
        body,
        grid=(grid,),
        in_specs=[
            pl.BlockSpec((BM, 32), lambda i: (i, 0)),
            pl.BlockSpec((BM, 1), lambda i: (i, 0)),
        ],
        out_specs=pl.BlockSpec((BM, 48), lambda i: (i, 0)),
        out_shape=jax.ShapeDtypeStruct((M, 48), jnp.float32),
    )(R, w.reshape(M, 1))


def _tc_pool(sc4, batch1):
    BN = 1000
    grid = N // BN

    def body(sc_ref, b_ref, o_ref, sums, cnt):
        i = pl.program_id(0)

        @pl.when(i == 0)
        def _():
            sums[...] = jnp.zeros_like(sums)
            cnt[...] = jnp.zeros_like(cnt)

        A = sc_ref[...]
        num = A[0] + A[1]
        hu = num[0, :, :32] / (num[0, :, 32:33] + 1e-16)
        hd = num[1, :, :32] / (num[1, :, 32:33] + 1e-16)
        hp = num[2, :, :32]
        x = jnp.maximum(hu + hd + hp, 0.0)
        b = b_ref[...]
        P = (b == lax.broadcasted_iota(jnp.int32, (BN, NB), 1)
             ).astype(jnp.float32)
        sums[...] += lax.dot_general(P, x, (((0,), (0,)), ((), ())),
                                     preferred_element_type=jnp.float32)
        cnt[...] += lax.dot_general(P, jnp.ones((BN, 1), jnp.float32),
                                    (((0,), (0,)), ((), ())),
                                    preferred_element_type=jnp.float32)

        @pl.when(i == grid - 1)
        def _():
            pooled = sums[...] / jnp.maximum(cnt[...], 1.0)
            z = pooled[:, :10]
            zm = jnp.max(z, axis=1, keepdims=True)
            ez = jnp.exp(z - zm)
            o_ref[...] = ez / jnp.sum(ez, axis=1, keepdims=True)

    return pl.pallas_call(
        body,
        grid=(grid,),
        in_specs=[
            pl.BlockSpec((NC, 3, BN, 48), lambda i: (0, 0, i, 0)),
            pl.BlockSpec((BN, 1), lambda i: (i, 0)),
        ],
        out_specs=pl.BlockSpec((NB, 10), lambda i: (0, 0)),
        out_shape=jax.ShapeDtypeStruct((NB, 10), jnp.float32),
        scratch_shapes=[
            pltpu.VMEM((NB, 32), jnp.float32),
            pltpu.VMEM((NB, 1), jnp.float32),
        ],
    )(sc4, batch1.reshape(N, 1))


def _pad_w(W, a):
    dout = W.shape[1]
    if dout < FW:
        W = jnp.pad(W, ((0, 0), (0, FW - dout)))
        a = jnp.pad(a, (0, FW - dout))
    return W, a.reshape(1, FW)


def kernel(x1, lu_idx, lu_vals, ld_idx, ld_vals, batch1,
           Wp1, Wg1, asrc1, adst1, Wp2, Wg2, asrc2, adst2,
           Wp3, Wg3, asrc3, adst3, Wp4, Wg4, asrc4, adst4):
    ia = jnp.concatenate([lu_idx[0], ld_idx[0]])
    ib = jnp.concatenate([lu_idx[1], ld_idx[1]])
    vals = jnp.concatenate([lu_vals, ld_vals])
    dstoff = jnp.concatenate([lu_idx[1], ld_idx[1] + N])
    gidx = jnp.concatenate([ia, ib + N])
    sidx = jnp.concatenate([lu_idx[1], ld_idx[1] + N, ia + 2 * N])

    diag = jnp.where(ia == ib, vals, 0.0)
    d = jnp.zeros((N,), jnp.float32).at[ia].add(diag)
    dinv = lax.rsqrt(jnp.where(d > 0, d, 1.0))
    dinvp = _pad16(dinv)
    GD1 = _sc_gather(dinvp, ia)
    GD2 = _sc_gather(dinvp, ib)
    pv = _tc_pv(vals.reshape(-1, 1), GD1, GD2)

    params = [(Wp1, Wg1, asrc1, adst1), (Wp2, Wg2, asrc2, adst2),
              (Wp3, Wg3, asrc3, adst3), (Wp4, Wg4, asrc4, adst4)]

    cur = x1
    first = True
    for (Wp, Wg, a_s, a_d) in params:
        Wgp, a_sp = _pad_w(Wg, a_s)
        Wpp, a_dp = _pad_w(Wp, a_d)
        T, es, ed = _tc_dense(cur, Wgp, Wpp, a_sp, a_dp, first)
        scal = jnp.concatenate([es, ed, jnp.zeros((N, 14), jnp.float32)], axis=1)
        GA = _sc_gather(scal, ia)
        GB = _sc_gather(scal, ib)
        e = _tc_logits(GA, GB)
        m = jax.ops.segment_max(e.reshape(-1), dstoff, num_segments=2 * N)
        m = jnp.where(jnp.isfinite(m), m, 0.0)
        GM = _sc_gather(_pad16(m), dstoff)
        ex = _tc_exp(e, GM)
        R = _sc_gather(T, gidx)
        w = jnp.concatenate([ex, pv], axis=0)
        S2 = _tc_scale(R, w)
        sc = _sc_scatter(S2, sidx, 3 * N).reshape(NC, 3, N, 48)
        cur = sc
        first = False

    return _tc_pool(cur, batch1)
